# baseline (speedup 1.0000x reference)
import functools

import jax
import jax.numpy as jnp
from jax import lax
from jax.experimental import pallas as pl
from jax.experimental.pallas import tpu as pltpu
from jax.experimental.pallas import tpu_sc as plsc

E = 800000
D = 64
NUM_ROWS = 5

NP = E // 2
PD = 2 * D
NPAIRS = NUM_ROWS * NUM_ROWS

NC = 2
NS = 16
NW = NC * NS

CHUNK = 400
SUBS = [(0, 128), (128, 128), (256, 128), (384, 16)]
NBUF = 2
N_CHUNKS = NP // CHUNK
BASE_CHUNKS = N_CHUNKS // NW
EXTRA = N_CHUNKS - BASE_CHUNKS * NW


def _embed_body(idx_hbm, table_hbm, out_hbm, table_sh,
                idx0, idx1, rows0, rows1, gsem0, gsem1, ssem0, ssem1):
    cid = lax.axis_index("c")
    sid = lax.axis_index("s")
    wid = cid * NS + sid

    has_extra = wid < EXTRA
    n_chunks = jnp.where(has_extra, BASE_CHUNKS + 1, BASE_CHUNKS)
    start_chunk = jnp.where(
        has_extra,
        wid * (BASE_CHUNKS + 1),
        EXTRA * (BASE_CHUNKS + 1) + (wid - EXTRA) * BASE_CHUNKS,
    )

    @pl.when(sid == 0)
    def _():
        pltpu.sync_copy(table_hbm, table_sh)

    plsc.subcore_barrier()

    idx_bufs = (idx0, idx1)
    rows_bufs = (rows0, rows1)
    gsems = (gsem0, gsem1)
    ssems = (ssem0, ssem1)

    def fire_gathers(c, b):
        rbase = c * CHUNK
        pltpu.sync_copy(idx_hbm.at[pl.ds(rbase, CHUNK)], idx_bufs[b])
        handles = []
        for (off, ln) in SUBS:
            handles.append(pltpu.async_copy(
                table_sh.at[idx_bufs[b].at[pl.ds(off, ln)]],
                rows_bufs[b].at[pl.ds(off, ln)],
                gsems[b],
            ))
        return handles

    def fire_store(c, b):
        obase = c * CHUNK
        pltpu.async_copy(rows_bufs[b], out_hbm.at[pl.ds(obase, CHUNK)],
                         ssems[b])

    def wait_store(c, b):
        obase = c * CHUNK
        pltpu.make_async_copy(
            rows_bufs[b], out_hbm.at[pl.ds(obase, CHUNK)], ssems[b]
        ).wait()

    def outer_body(i, carry):
        all_handles = []
        for b in range(NBUF):
            c = start_chunk + i * NBUF + b

            @pl.when(i > 0)
            def _():
                wait_store(c, b)

            all_handles.append(fire_gathers(c, b))
        for b in range(NBUF):
            c = start_chunk + i * NBUF + b
            for h in all_handles[b]:
                h.wait()
            fire_store(c, b)
        return carry

    n_outer = n_chunks // NBUF
    lax.fori_loop(0, n_outer, outer_body, 0)

    @pl.when(n_chunks % NBUF == 1)
    def _():
        c = start_chunk + n_chunks - 1
        wait_store(c, 0)
        handles = fire_gathers(c, 0)
        for h in handles:
            h.wait()
        fire_store(c, 0)

    for b in range(NBUF):
        wait_store(start_chunk, b)


def kernel(bond_types, table):
    bt = bond_types.astype(jnp.int32).reshape(NP, 2)
    pidx = bt[:, 0] * NUM_ROWS + bt[:, 1]
    tpairs = jnp.concatenate(
        [jnp.repeat(table, NUM_ROWS, axis=0), jnp.tile(table, (NUM_ROWS, 1))],
        axis=1,
    )
    mesh = plsc.VectorSubcoreMesh(core_axis_name="c", subcore_axis_name="s")
    kern = functools.partial(
        pl.kernel,
        out_type=jax.ShapeDtypeStruct((NP, PD), jnp.float32),
        mesh=mesh,
        scratch_types=[
            pltpu.VMEM_SHARED((NPAIRS, PD), jnp.float32),
            pltpu.VMEM((CHUNK,), jnp.int32),
            pltpu.VMEM((CHUNK,), jnp.int32),
            pltpu.VMEM((CHUNK, PD), jnp.float32),
            pltpu.VMEM((CHUNK, PD), jnp.float32),
            pltpu.SemaphoreType.DMA,
            pltpu.SemaphoreType.DMA,
            pltpu.SemaphoreType.DMA,
            pltpu.SemaphoreType.DMA,
        ],
        compiler_params=pltpu.CompilerParams(use_tc_tiling_on_sc=False),
    )(_embed_body)
    out_pairs = kern(pidx, tpairs)
    return out_pairs.reshape(E, D)


# speedup vs baseline: 1.5098x; 1.5098x over previous
import functools

import jax
import jax.numpy as jnp
from jax import lax
from jax.experimental import pallas as pl
from jax.experimental.pallas import tpu as pltpu
from jax.experimental.pallas import tpu_sc as plsc

E = 800000
D = 64
NUM_ROWS = 5

NC = 2
NS = 16
NW = NC * NS

CHUNK = 1000
SUBS = [(0, 128), (128, 128), (256, 128), (384, 128),
        (512, 128), (640, 128), (768, 128), (896, 104)]
NBUF = 2
ROWS_PER_W = E // NW
CHUNKS_PER_W = ROWS_PER_W // CHUNK


def _embed_body(idx_hbm, table_hbm, out_hbm, table_sh,
                idx0, idx1, rows0, rows1, gsem0, gsem1, ssem0, ssem1):
    cid = lax.axis_index("c")
    sid = lax.axis_index("s")
    wid = cid * NS + sid
    out_base = wid * ROWS_PER_W

    @pl.when(sid == 0)
    def _():
        pltpu.sync_copy(table_hbm, table_sh)

    plsc.subcore_barrier()

    idx_bufs = (idx0, idx1)
    rows_bufs = (rows0, rows1)
    gsems = (gsem0, gsem1)
    ssems = (ssem0, ssem1)

    def fire_gathers(c, b):
        rbase = out_base + c * CHUNK
        pltpu.sync_copy(idx_hbm.at[pl.ds(rbase, CHUNK)], idx_bufs[b])
        handles = []
        for (off, ln) in SUBS:
            handles.append(pltpu.async_copy(
                table_sh.at[idx_bufs[b].at[pl.ds(off, ln)]],
                rows_bufs[b].at[pl.ds(off, ln)],
                gsems[b],
            ))
        return handles

    def fire_store(c, b):
        obase = out_base + c * CHUNK
        pltpu.async_copy(rows_bufs[b], out_hbm.at[pl.ds(obase, CHUNK)],
                         ssems[b])

    def wait_store(c, b):
        obase = out_base + c * CHUNK
        pltpu.make_async_copy(
            rows_bufs[b], out_hbm.at[pl.ds(obase, CHUNK)], ssems[b]
        ).wait()

    def outer_body(i, carry):
        all_handles = []
        for b in range(NBUF):
            c = i * NBUF + b

            @pl.when(i > 0)
            def _():
                wait_store(c, b)

            all_handles.append(fire_gathers(c, b))
        for b in range(NBUF):
            c = i * NBUF + b
            for h in all_handles[b]:
                h.wait()
            fire_store(c, b)
        return carry

    n_outer = CHUNKS_PER_W // NBUF
    lax.fori_loop(0, n_outer, outer_body, 0)
    for b in range(NBUF):
        c = (n_outer - 1) * NBUF + b
        wait_store(c, b)
    for c in range(n_outer * NBUF, CHUNKS_PER_W):
        handles = fire_gathers(c, 0)
        for h in handles:
            h.wait()
        fire_store(c, 0)
        wait_store(c, 0)


def kernel(bond_types, table):
    idx1d = bond_types
    mesh = plsc.VectorSubcoreMesh(core_axis_name="c", subcore_axis_name="s")
    kern = functools.partial(
        pl.kernel,
        out_type=jax.ShapeDtypeStruct((E, D), jnp.float32),
        mesh=mesh,
        scratch_types=[
            pltpu.VMEM_SHARED((NUM_ROWS, D), jnp.float32),
            pltpu.VMEM((CHUNK,), jnp.int32),
            pltpu.VMEM((CHUNK,), jnp.int32),
            pltpu.VMEM((CHUNK, D), jnp.float32),
            pltpu.VMEM((CHUNK, D), jnp.float32),
            pltpu.SemaphoreType.DMA,
            pltpu.SemaphoreType.DMA,
            pltpu.SemaphoreType.DMA,
            pltpu.SemaphoreType.DMA,
        ],
        compiler_params=pltpu.CompilerParams(use_tc_tiling_on_sc=False),
    )(_embed_body)
    return kern(idx1d, table)


# speedup vs baseline: 2.3762x; 1.5738x over previous
import functools

import jax
import jax.numpy as jnp
from jax import lax
from jax.experimental import pallas as pl
from jax.experimental.pallas import tpu as pltpu
from jax.experimental.pallas import tpu_sc as plsc

E = 800000
D = 64
NUM_ROWS = 5

NC = 2
NS = 16
NW = NC * NS

CHUNK = 200
SUBS = [(0, 128), (128, 72)]
NBUF = 2
ROWS_PER_W = E // NW
CHUNKS_PER_W = ROWS_PER_W // CHUNK


def _embed_body(idx_hbm, table_hbm, out_hbm, table_sh,
                idx0, idx1, rows0, rows1, gsem0, gsem1, ssem0, ssem1):
    cid = lax.axis_index("c")
    sid = lax.axis_index("s")
    wid = cid * NS + sid
    out_base = wid * ROWS_PER_W

    @pl.when(sid == 0)
    def _():
        pltpu.sync_copy(table_hbm, table_sh)

    plsc.subcore_barrier()

    idx_bufs = (idx0, idx1)
    rows_bufs = (rows0, rows1)
    gsems = (gsem0, gsem1)
    ssems = (ssem0, ssem1)

    def fire_gathers(c, b):
        rbase = out_base + c * CHUNK
        pltpu.sync_copy(idx_hbm.at[pl.ds(rbase, CHUNK)], idx_bufs[b])
        handles = []
        for (off, ln) in SUBS:
            handles.append(pltpu.async_copy(
                table_sh.at[idx_bufs[b].at[pl.ds(off, ln)]],
                rows_bufs[b].at[pl.ds(off, ln)],
                gsems[b],
            ))
        return handles

    def fire_store(c, b):
        obase = out_base + c * CHUNK
        pltpu.async_copy(rows_bufs[b], out_hbm.at[pl.ds(obase, CHUNK)],
                         ssems[b])

    def wait_store(c, b):
        obase = out_base + c * CHUNK
        pltpu.make_async_copy(
            rows_bufs[b], out_hbm.at[pl.ds(obase, CHUNK)], ssems[b]
        ).wait()

    def outer_body(i, carry):
        all_handles = []
        for b in range(NBUF):
            c = i * NBUF + b

            @pl.when(i > 0)
            def _():
                wait_store(c, b)

            all_handles.append(fire_gathers(c, b))
        for b in range(NBUF):
            c = i * NBUF + b
            for h in all_handles[b]:
                h.wait()
            fire_store(c, b)
        return carry

    n_outer = CHUNKS_PER_W // NBUF
    lax.fori_loop(0, n_outer, outer_body, 0)
    for b in range(NBUF):
        c = (n_outer - 1) * NBUF + b
        wait_store(c, b)
    for c in range(n_outer * NBUF, CHUNKS_PER_W):
        handles = fire_gathers(c, 0)
        for h in handles:
            h.wait()
        fire_store(c, 0)
        wait_store(c, 0)


def kernel(bond_types, table):
    idx1d = bond_types
    table2 = jnp.concatenate([table, table], axis=1)
    mesh = plsc.VectorSubcoreMesh(core_axis_name="c", subcore_axis_name="s")
    kern = functools.partial(
        pl.kernel,
        out_type=jax.ShapeDtypeStruct((E, 2 * D), jnp.float32),
        mesh=mesh,
        scratch_types=[
            pltpu.VMEM_SHARED((NUM_ROWS, 2 * D), jnp.float32),
            pltpu.VMEM((CHUNK,), jnp.int32),
            pltpu.VMEM((CHUNK,), jnp.int32),
            pltpu.VMEM((CHUNK, 2 * D), jnp.float32),
            pltpu.VMEM((CHUNK, 2 * D), jnp.float32),
            pltpu.SemaphoreType.DMA,
            pltpu.SemaphoreType.DMA,
            pltpu.SemaphoreType.DMA,
            pltpu.SemaphoreType.DMA,
        ],
        compiler_params=pltpu.CompilerParams(use_tc_tiling_on_sc=True),
    )(_embed_body)
    return kern(idx1d, table2)[:, :D]
